# baseline (speedup 1.0000x reference)
import functools

import jax
import jax.numpy as jnp
from jax import lax
from jax.experimental import pallas as pl
from jax.experimental.pallas import tpu as pltpu
from jax.experimental.pallas import tpu_sc as plsc

NTIME, NBATCH, K = 100, 1024, 128
NPAD = 128
NC, NS = 2, 16
RB, CB = 4, 8
KW = K // RB
CW = NBATCH // CB
LANES = 16
NG = CW // LANES


def _interp_body(times_hbm, values_hbm, t_hbm, out_hbm,
                 times_v, values_v, t_v, out_v):
    wid = lax.axis_index("s") * NC + lax.axis_index("c")
    rb = wid // CB
    cb = wid - rb * CB
    r0 = rb * KW
    c0 = cb * CW
    pltpu.sync_copy(times_hbm.at[:, pl.ds(c0, CW)], times_v.at[pl.ds(0, NTIME), :])
    pltpu.sync_copy(values_hbm.at[:, pl.ds(c0, CW)], values_v)
    pltpu.sync_copy(t_hbm.at[pl.ds(r0, KW), pl.ds(c0, CW)], t_v)

    inf16 = jnp.full((LANES,), jnp.inf, jnp.float32)
    for r in range(NTIME, NPAD):
        for g in range(NG):
            times_v[r, pl.ds(g * LANES, LANES)] = inf16

    for g in range(NG):
        sl = pl.ds(g * LANES, LANES)
        coff = lax.iota(jnp.int32, LANES) + (g * LANES)
        t63, t31, t95, t15, t47, t79, t111 = (
            times_v[r, sl] for r in (63, 31, 95, 15, 47, 79, 111))
        s4 = tuple(times_v[r, sl] for r in (7, 23, 39, 55, 71, 87, 103, 119))

        @plsc.parallel_loop(0, KW, step=1, unroll=2)
        def query_body(k):
            tq = t_v[k, sl]
            b64 = t63 <= tq
            pos = jnp.where(b64, 64, 0)
            b32 = jnp.where(b64, t95, t31) <= tq
            pos = jnp.where(b32, pos + 32, pos)
            m3 = jnp.where(b32,
                           jnp.where(b64, t111, t47),
                           jnp.where(b64, t79, t15))
            b16 = m3 <= tq
            pos = jnp.where(b16, pos + 16, pos)
            m4 = jnp.where(b16,
                           jnp.where(b32,
                                     jnp.where(b64, s4[7], s4[3]),
                                     jnp.where(b64, s4[5], s4[1])),
                           jnp.where(b32,
                                     jnp.where(b64, s4[6], s4[2]),
                                     jnp.where(b64, s4[4], s4[0])))
            pos = jnp.where(m4 <= tq, pos + 8, pos)
            for w in (4, 2, 1):
                gk = plsc.load_gather(times_v, [pos + (w - 1), coff])
                pos = jnp.where(gk <= tq, pos + w, pos)
            is0 = (pos == 0) | (pos == NTIME)
            isl = jnp.where(is0, NTIME - 2, pos - 1)
            va = plsc.load_gather(values_v, [isl, coff])
            vb = plsc.load_gather(values_v, [isl + 1, coff])
            ta = plsc.load_gather(times_v, [isl, coff])
            tb = plsc.load_gather(times_v, [isl + 1, coff])
            gv = jnp.where(is0, vb, va)
            gt = jnp.where(is0, tb, ta)
            out_v[k, sl] = gv + ((vb - va) / (tb - ta)) * (tq - gt)

    pltpu.sync_copy(out_v, out_hbm.at[pl.ds(r0, KW), pl.ds(c0, CW)])


@jax.jit
def _run(times, values, t):
    mesh = plsc.VectorSubcoreMesh(core_axis_name="c", subcore_axis_name="s")
    f = functools.partial(
        pl.kernel,
        mesh=mesh,
        compiler_params=pltpu.CompilerParams(
            needs_layout_passes=False, use_tc_tiling_on_sc=True,
            skip_device_barrier=True),
        out_type=jax.ShapeDtypeStruct((K, NBATCH), jnp.float32),
        scratch_types=[
            pltpu.VMEM((NPAD, CW), jnp.float32),
            pltpu.VMEM((NTIME, CW), jnp.float32),
            pltpu.VMEM((KW, CW), jnp.float32),
            pltpu.VMEM((KW, CW), jnp.float32),
        ],
    )(_interp_body)
    return f(times, values, t)


def kernel(times, values, t):
    return _run(times, values, t)


# speedup vs baseline: 1.0396x; 1.0396x over previous
import functools

import jax
import jax.numpy as jnp
from jax import lax
from jax.experimental import pallas as pl
from jax.experimental.pallas import tpu as pltpu
from jax.experimental.pallas import tpu_sc as plsc

NTIME, NBATCH, K = 100, 1024, 128
NPAD = 128
NC, NS = 2, 16
RB, CB = 4, 8
KW = K // RB
CW = NBATCH // CB
LANES = 16
NG = CW // LANES


def _interp_body(times_hbm, values_hbm, t_hbm, out_hbm,
                 times_v, values_v, t_v, out_v, sem_t, sem_v, sem_q):
    wid = lax.axis_index("s") * NC + lax.axis_index("c")
    rb = wid // CB
    cb = wid - rb * CB
    r0 = rb * KW
    c0 = cb * CW
    cp_t = pltpu.async_copy(
        times_hbm.at[:, pl.ds(c0, CW)], times_v.at[pl.ds(0, NTIME), :], sem_t)
    cp_v = pltpu.async_copy(values_hbm.at[:, pl.ds(c0, CW)], values_v, sem_v)
    cp_q = pltpu.async_copy(
        t_hbm.at[pl.ds(r0, KW), pl.ds(c0, CW)], t_v, sem_q)

    inf16 = jnp.full((LANES,), jnp.inf, jnp.float32)
    for r in range(NTIME, NPAD):
        for g in range(NG):
            times_v[r, pl.ds(g * LANES, LANES)] = inf16
    cp_t.wait()
    cp_v.wait()
    cp_q.wait()

    for g in range(NG):
        sl = pl.ds(g * LANES, LANES)
        coff = lax.iota(jnp.int32, LANES) + (g * LANES)
        t63, t31, t95, t15, t47, t79, t111 = (
            times_v[r, sl] for r in (63, 31, 95, 15, 47, 79, 111))
        s4 = tuple(times_v[r, sl] for r in (7, 23, 39, 55, 71, 87, 103, 119))

        @plsc.parallel_loop(0, KW, step=1, unroll=2)
        def query_body(k):
            tq = t_v[k, sl]
            b64 = t63 <= tq
            pos = jnp.where(b64, 64, 0)
            b32 = jnp.where(b64, t95, t31) <= tq
            pos = jnp.where(b32, pos + 32, pos)
            m3 = jnp.where(b32,
                           jnp.where(b64, t111, t47),
                           jnp.where(b64, t79, t15))
            b16 = m3 <= tq
            pos = jnp.where(b16, pos + 16, pos)
            m4 = jnp.where(b16,
                           jnp.where(b32,
                                     jnp.where(b64, s4[7], s4[3]),
                                     jnp.where(b64, s4[5], s4[1])),
                           jnp.where(b32,
                                     jnp.where(b64, s4[6], s4[2]),
                                     jnp.where(b64, s4[4], s4[0])))
            pos = jnp.where(m4 <= tq, pos + 8, pos)
            for w in (4, 2, 1):
                gk = plsc.load_gather(times_v, [pos + (w - 1), coff])
                pos = jnp.where(gk <= tq, pos + w, pos)
            is0 = (pos == 0) | (pos == NTIME)
            isl = jnp.where(is0, NTIME - 2, pos - 1)
            va = plsc.load_gather(values_v, [isl, coff])
            vb = plsc.load_gather(values_v, [isl + 1, coff])
            ta = plsc.load_gather(times_v, [isl, coff])
            tb = plsc.load_gather(times_v, [isl + 1, coff])
            gv = jnp.where(is0, vb, va)
            gt = jnp.where(is0, tb, ta)
            out_v[k, sl] = gv + ((vb - va) / (tb - ta)) * (tq - gt)

    pltpu.sync_copy(out_v, out_hbm.at[pl.ds(r0, KW), pl.ds(c0, CW)])


@jax.jit
def _run(times, values, t):
    mesh = plsc.VectorSubcoreMesh(core_axis_name="c", subcore_axis_name="s")
    f = functools.partial(
        pl.kernel,
        mesh=mesh,
        compiler_params=pltpu.CompilerParams(
            needs_layout_passes=False, use_tc_tiling_on_sc=True,
            skip_device_barrier=True),
        out_type=jax.ShapeDtypeStruct((K, NBATCH), jnp.float32),
        scratch_types=[
            pltpu.VMEM((NPAD, CW), jnp.float32),
            pltpu.VMEM((NTIME, CW), jnp.float32),
            pltpu.VMEM((KW, CW), jnp.float32),
            pltpu.VMEM((KW, CW), jnp.float32),
            pltpu.SemaphoreType.DMA,
            pltpu.SemaphoreType.DMA,
            pltpu.SemaphoreType.DMA,
        ],
    )(_interp_body)
    return f(times, values, t)


def kernel(times, values, t):
    return _run(times, values, t)


# speedup vs baseline: 1.1813x; 1.1362x over previous
import functools

import jax
import jax.numpy as jnp
from jax import lax
from jax.experimental import pallas as pl
from jax.experimental.pallas import tpu as pltpu
from jax.experimental.pallas import tpu_sc as plsc

NTIME, NBATCH, K = 100, 1024, 128
NPAD = 128
NC, NS = 2, 16
NW = NC * NS
CW = NBATCH // NW
LANES = 16


def _interp_body(times_hbm, values_hbm, t_hbm, out_hbm,
                 times_v, values_v, slopes_v, t_v, out_v,
                 sem_t, sem_v, sem_q):
    wid = lax.axis_index("s") * NC + lax.axis_index("c")
    c0 = wid * CW
    cp_t = pltpu.async_copy(
        times_hbm.at[:, pl.ds(c0, CW)], times_v.at[pl.ds(0, NTIME), :], sem_t)
    cp_v = pltpu.async_copy(values_hbm.at[:, pl.ds(c0, CW)], values_v, sem_v)
    cp_q = pltpu.async_copy(t_hbm.at[:, pl.ds(c0, CW)], t_v, sem_q)

    inf16 = jnp.full((LANES,), jnp.inf, jnp.float32)
    for r in range(NTIME, NPAD):
        for g in range(CW // LANES):
            times_v[r, pl.ds(g * LANES, LANES)] = inf16
    cp_t.wait()
    cp_v.wait()
    cp_q.wait()

    @plsc.parallel_loop(0, NTIME - 1, step=1, unroll=4)
    def slope_body(i):
        for g in range(CW // LANES):
            v0 = values_v[i, pl.ds(g * LANES, LANES)]
            v1 = values_v[i + 1, pl.ds(g * LANES, LANES)]
            t0 = times_v[i, pl.ds(g * LANES, LANES)]
            t1 = times_v[i + 1, pl.ds(g * LANES, LANES)]
            slopes_v[i, pl.ds(g * LANES, LANES)] = (v1 - v0) / (t1 - t0)

    NG = CW // LANES
    pre = []
    for g in range(NG):
        sl = pl.ds(g * LANES, LANES)
        pre.append((
            tuple(times_v[r, sl] for r in (63, 31, 95, 15, 47, 79, 111)),
            tuple(times_v[r, sl] for r in (7, 23, 39, 55, 71, 87, 103, 119)),
        ))

    @plsc.parallel_loop(0, K, step=1, unroll=2)
    def query_body(k):
        for g in range(NG):
            coff = lax.iota(jnp.int32, LANES) + (g * LANES)
            (t63, t31, t95, t15, t47, t79, t111), s4 = pre[g]
            tq = t_v[k, pl.ds(g * LANES, LANES)]
            b64 = t63 <= tq
            pos = jnp.where(b64, 64, 0)
            b32 = jnp.where(b64, t95, t31) <= tq
            pos = jnp.where(b32, pos + 32, pos)
            m3 = jnp.where(b32,
                           jnp.where(b64, t111, t47),
                           jnp.where(b64, t79, t15))
            b16 = m3 <= tq
            pos = jnp.where(b16, pos + 16, pos)
            m4 = jnp.where(b16,
                           jnp.where(b32,
                                     jnp.where(b64, s4[7], s4[3]),
                                     jnp.where(b64, s4[5], s4[1])),
                           jnp.where(b32,
                                     jnp.where(b64, s4[6], s4[2]),
                                     jnp.where(b64, s4[4], s4[0])))
            pos = jnp.where(m4 <= tq, pos + 8, pos)
            for w in (4, 2, 1):
                gk = plsc.load_gather(times_v, [pos + (w - 1), coff])
                pos = jnp.where(gk <= tq, pos + w, pos)
            is0 = (pos == 0) | (pos == NTIME)
            iv = jnp.where(is0, NTIME - 1, pos - 1)
            isl = jnp.where(is0, NTIME - 2, pos - 1)
            gv = plsc.load_gather(values_v, [iv, coff])
            gt = plsc.load_gather(times_v, [iv, coff])
            gs = plsc.load_gather(slopes_v, [isl, coff])
            out_v[k, pl.ds(g * LANES, LANES)] = gv + gs * (tq - gt)
    pltpu.sync_copy(out_v, out_hbm.at[:, pl.ds(c0, CW)])


@jax.jit
def _run(times, values, t):
    mesh = plsc.VectorSubcoreMesh(core_axis_name="c", subcore_axis_name="s")
    f = functools.partial(
        pl.kernel,
        mesh=mesh,
        compiler_params=pltpu.CompilerParams(
            needs_layout_passes=False, use_tc_tiling_on_sc=False,
            skip_device_barrier=True),
        out_type=jax.ShapeDtypeStruct((K, NBATCH), jnp.float32),
        scratch_types=[
            pltpu.VMEM((NPAD, CW), jnp.float32),
            pltpu.VMEM((NTIME, CW), jnp.float32),
            pltpu.VMEM((NTIME - 1, CW), jnp.float32),
            pltpu.VMEM((K, CW), jnp.float32),
            pltpu.VMEM((K, CW), jnp.float32),
            pltpu.SemaphoreType.DMA,
            pltpu.SemaphoreType.DMA,
            pltpu.SemaphoreType.DMA,
        ],
    )(_interp_body)
    return f(times, values, t)


def kernel(times, values, t):
    return _run(times, values, t)
